# baseline (speedup 1.0000x reference)
import functools

import jax
import jax.numpy as jnp
from jax import lax
from jax.experimental import pallas as pl
from jax.experimental.pallas import tpu as pltpu
from jax.experimental.pallas import tpu_sc as plsc

_NC = 2
_NS = 16
_NW = _NC * _NS

_CHUNK = 128
_GATHERS = 4
_BLOCK = _CHUNK * _GATHERS
_NBUF = 2


@functools.lru_cache(maxsize=None)
def _build(B, V, D):
    assert B % (_NW * _BLOCK * _NBUF) == 0
    per_w = B // _NW
    nblk = per_w // _BLOCK
    block_bytes = _BLOCK * D * 4

    mesh = plsc.VectorSubcoreMesh(core_axis_name="c", subcore_axis_name="s")

    @functools.partial(
        pl.kernel,
        out_type=jax.ShapeDtypeStruct((B, D), jnp.float32),
        mesh=mesh,
        scratch_types=[
            pltpu.VMEM((_NBUF, _BLOCK), jnp.int32),
            pltpu.VMEM((_NBUF, _BLOCK, D), jnp.float32),
            pltpu.SemaphoreType.DMA((_NBUF,)),
        ],
        compiler_params=pltpu.CompilerParams(use_tc_tiling_on_sc=False),
    )
    def emb(idx_hbm, table_hbm, out_hbm, idx_v, rows_v, gsem):
        wid = lax.axis_index("s") * _NC + lax.axis_index("c")
        base = wid * per_w

        def fire(blk, p):
            off = pl.multiple_of(base + blk * _BLOCK, _BLOCK)
            pltpu.sync_copy(idx_hbm.at[pl.ds(off, _BLOCK)], idx_v.at[p])
            for j in range(_GATHERS):
                pltpu.async_copy(
                    table_hbm.at[idx_v.at[p, pl.ds(j * _CHUNK, _CHUNK)]],
                    rows_v.at[p, pl.ds(j * _CHUNK, _CHUNK)],
                    gsem.at[p],
                )

        for p in range(_NBUF):
            fire(p, p)

        def body(h, carry):
            for p in range(_NBUF):
                blk = h * _NBUF + p
                pltpu.make_async_copy(
                    out_hbm.at[pl.ds(0, _BLOCK)], rows_v.at[p], gsem.at[p]
                ).wait()
                off = pl.multiple_of(base + blk * _BLOCK, _BLOCK)
                pltpu.sync_copy(rows_v.at[p], out_hbm.at[pl.ds(off, _BLOCK)])

                @pl.when(blk + _NBUF < nblk)
                def _():
                    fire(blk + _NBUF, p)

            return carry

        lax.fori_loop(0, nblk // _NBUF, body, 0)

    return emb


def kernel(visit_order, pos_embed):
    R, S = visit_order.shape
    V, D = pos_embed.shape
    B = R * S
    idx = visit_order.reshape(B).astype(jnp.int32)
    out = _build(B, V, D)(idx, pos_embed)
    return out.reshape(R, S, D)
